# Initial kernel scaffold; baseline (speedup 1.0000x reference)
import jax
import jax.numpy as jnp
from jax.experimental import pallas as pl


def kernel(pedestal_table, spatial_embeddings, pedestals):
    raise NotImplementedError("write your pallas kernel here")


# speedup vs baseline: 2.1168x; 2.1168x over previous
import functools
import jax
import jax.numpy as jnp
from jax import lax
from jax.experimental import pallas as pl
from jax.experimental.pallas import tpu as pltpu, tpu_sc as plsc

N = 1048576
OUT_F = 6
PED_F = 4
SPA_F = 2

_info = plsc.get_sparse_core_info()
NC = _info.num_cores
NS = _info.num_subcores
L = _info.num_lanes
NW = NC * NS

CHUNK = N // NW
S = 4096
STEPS = CHUNK // S
BLOCKS = S // L


def _body(table_hbm, spat_hbm, ped_hbm, out_hbm, tbl_v, ped_v, spat_v, out_v):
    wid = lax.axis_index("s") * NC + lax.axis_index("c")
    base = wid * CHUNK

    pltpu.sync_copy(table_hbm, tbl_v)

    lane = lax.iota(jnp.int32, L)
    sp_off = (lax.shift_right_logical(lane, 1) * OUT_F
              + PED_F + jnp.bitwise_and(lane, 1))

    def step_fn(step, _):
        off = base + step * S
        pltpu.sync_copy(ped_hbm.at[pl.ds(off, S)], ped_v)
        pltpu.sync_copy(spat_hbm.at[pl.ds(SPA_F * off, SPA_F * S)], spat_v)

        def blk_fn(b, _):
            row0 = b * L
            ped_vec = ped_v[pl.ds(row0, L)]
            tidx = ped_vec * PED_F
            obase = row0 * OUT_F + lane * OUT_F
            for j in range(PED_F):
                col = plsc.load_gather(tbl_v, [tidx + j])
                plsc.store_scatter(out_v, [obase + j], col)
            sbase = b * (2 * L)
            va = spat_v[pl.ds(sbase, L)]
            vb = spat_v[pl.ds(sbase + L, L)]
            plsc.store_scatter(out_v, [row0 * OUT_F + sp_off], va)
            plsc.store_scatter(out_v, [(row0 + 8) * OUT_F + sp_off], vb)
            return _

        lax.fori_loop(0, BLOCKS, blk_fn, None)
        pltpu.sync_copy(out_v, out_hbm.at[pl.ds(OUT_F * off, OUT_F * S)])
        return _

    lax.fori_loop(0, STEPS, step_fn, None)


def kernel(pedestal_table, spatial_embeddings, pedestals):
    mesh = plsc.VectorSubcoreMesh(core_axis_name="c", subcore_axis_name="s")
    tbl_flat = pedestal_table.reshape(16 * PED_F)
    spat_flat = spatial_embeddings.reshape(N * SPA_F)
    k = functools.partial(
        pl.kernel,
        mesh=mesh,
        out_type=jax.ShapeDtypeStruct((N * OUT_F,), jnp.float32),
        scratch_types=[
            pltpu.VMEM((16 * PED_F,), jnp.float32),
            pltpu.VMEM((S,), jnp.int32),
            pltpu.VMEM((S * SPA_F,), jnp.float32),
            pltpu.VMEM((S * OUT_F,), jnp.float32),
        ],
        compiler_params=pltpu.CompilerParams(needs_layout_passes=False),
    )(_body)
    return k(tbl_flat, spat_flat, pedestals).reshape(N, OUT_F)
